# baseline (speedup 1.0000x reference)
import functools

import jax
import jax.numpy as jnp
from jax import lax
from jax.experimental import pallas as pl
from jax.experimental.pallas import tpu as pltpu
from jax.experimental.pallas import tpu_sc as plsc

B = 4096
S = 200
D = 64
OUT = 64
NC = 2
NS = 16
NW = NC * NS
BPW = B // NW
NCHUNK = 2
CH = S // NCHUNK
NLANE = 16
NJ = D // NLANE


def _pooled_body(tok_hbm, table_hbm, out_hbm, idx_v, rows_v, pooled_v, sem0, sem1):
    wid = lax.axis_index("s") * NC + lax.axis_index("c")
    base = wid * BPW
    pltpu.sync_copy(tok_hbm.at[pl.ds(base, BPW)], idx_v)

    sems = (sem0, sem1)

    def issue(i, nb):
        for c in range(NCHUNK):
            pltpu.async_copy(
                table_hbm.at[idx_v.at[i, c]],
                rows_v.at[nb, pl.ds(c * CH, CH)],
                sems[nb],
            )

    def drain(i, nb):
        for c in range(NCHUNK):
            pltpu.make_async_copy(
                table_hbm.at[idx_v.at[i, c]],
                rows_v.at[nb, pl.ds(c * CH, CH)],
                sems[nb],
            ).wait()

    issue(0, 0)
    issue(1, 1)

    def group_body(g, carry):
        for nb in range(2):
            i = g * 2 + nb
            drain(i, nb)

            def acc_body(s_, accs):
                return tuple(
                    accs[j] + rows_v[nb, s_, pl.ds(j * NLANE, NLANE)]
                    for j in range(NJ)
                )

            accs = lax.fori_loop(
                0, S, acc_body,
                tuple(jnp.zeros((NLANE,), jnp.float32) for _ in range(NJ)),
            )

            @pl.when(i + 2 < BPW)
            def _():
                issue(i + 2, nb)

            for j in range(NJ):
                pooled_v[i, pl.ds(j * NLANE, NLANE)] = accs[j] * (1.0 / S)
        return carry

    lax.fori_loop(0, BPW // 2, group_body, 0)
    pltpu.sync_copy(pooled_v, out_hbm.at[pl.ds(base, BPW)])


_pooled = functools.partial(
    pl.kernel,
    out_type=jax.ShapeDtypeStruct((B, D), jnp.float32),
    mesh=plsc.VectorSubcoreMesh(core_axis_name="c", subcore_axis_name="s"),
    scratch_types=[
        pltpu.VMEM((BPW, NCHUNK, CH), jnp.int32),
        pltpu.VMEM((2, S, D), jnp.float32),
        pltpu.VMEM((BPW, D), jnp.float32),
        pltpu.SemaphoreType.DMA,
        pltpu.SemaphoreType.DMA,
    ],
    compiler_params=pltpu.CompilerParams(use_tc_tiling_on_sc=False),
)(_pooled_body)


def _linear_body(x_ref, w_ref, b_ref, o_ref):
    o_ref[...] = (
        lax.dot_general(
            x_ref[...], w_ref[...],
            (((1,), (1,)), ((), ())),
            preferred_element_type=jnp.float32,
        )
        + b_ref[...]
    )


_linear = pl.pallas_call(
    _linear_body,
    out_shape=jax.ShapeDtypeStruct((B, OUT), jnp.float32),
    grid=(8,),
    in_specs=[
        pl.BlockSpec((B // 8, D), lambda i: (i, 0)),
        pl.BlockSpec((OUT, D), lambda i: (0, 0)),
        pl.BlockSpec((1, OUT), lambda i: (0, 0)),
    ],
    out_specs=pl.BlockSpec((B // 8, OUT), lambda i: (i, 0)),
)


def kernel(token_ids, table, W, b):
    tok3 = token_ids.astype(jnp.int32).reshape(B, NCHUNK, CH)
    pooled = _pooled(tok3, table)
    return _linear(pooled, W, b.reshape(1, OUT))
